# baseline (speedup 1.0000x reference)
import dataclasses

import jax
import jax.numpy as jnp
from jax import lax
from jax.experimental import pallas as pl
from jax.experimental.pallas import tpu as pltpu
from jax.experimental.pallas import tpu_sc as plsc

N = 10000
D = 128
E = 320000

NUM_CORES = 2
NUM_SUBCORES = 16
CHUNK = 320
NCH = 32
EPC_PAD = NUM_SUBCORES * NCH * CHUNK
PAD = EPC_PAD - E // NUM_CORES
OWN_ROWS = 1000
ZROWS = 40


def _sc_aggregate(x, src, dst, w):
    mesh = plsc.VectorSubcoreMesh(core_axis_name="c", subcore_axis_name="s")

    @pl.kernel(
        out_type=jax.ShapeDtypeStruct((NUM_CORES, N, D), jnp.float32),
        mesh=mesh,
        scratch_types=[
            pltpu.VMEM_SHARED((N, D), jnp.float32),
            pltpu.VMEM((CHUNK, D), jnp.float32),
            pltpu.VMEM((CHUNK,), jnp.int32),
            pltpu.VMEM((CHUNK,), jnp.int32),
            pltpu.VMEM((CHUNK,), jnp.float32),
            pltpu.SemaphoreType.DMA,
        ],
    )
    def agg_kernel(x_hbm, src_hbm, dst_hbm, w_hbm, out_hbm,
                   acc, rows_v, src_v, dst_v, w_v, sem0):
        cid = lax.axis_index("c")
        sid = lax.axis_index("s")

        @pl.when(sid < N // OWN_ROWS)
        def _():
            zero16 = jnp.zeros((16,), jnp.float32)
            for r in range(ZROWS):
                for j in range(D // 16):
                    rows_v[r, pl.ds(j * 16, 16)] = zero16
            base_row = pl.multiple_of(sid * OWN_ROWS, 8)

            @pl.loop(0, OWN_ROWS, step=ZROWS)
            def _(t):
                pltpu.sync_copy(rows_v.at[pl.ds(0, ZROWS)],
                                acc.at[pl.ds(base_row + t, ZROWS)])

        plsc.subcore_barrier()

        cbase = cid * EPC_PAD

        @pl.loop(sid, NCH * NUM_SUBCORES, step=NUM_SUBCORES)
        def _(c):
            b = pl.multiple_of(cbase + c * CHUNK, 8)
            pltpu.sync_copy(src_hbm.at[pl.ds(b, CHUNK)], src_v)
            pltpu.sync_copy(dst_hbm.at[pl.ds(b, CHUNK)], dst_v)
            pltpu.sync_copy(w_hbm.at[pl.ds(b, CHUNK)], w_v)
            pltpu.async_copy(x_hbm.at[src_v], rows_v, sem0).wait()

            @pl.loop(0, CHUNK, step=16)
            def _(g):
                wg = w_v[pl.ds(g, 16)]
                for k in range(16):
                    wi = wg[k]
                    for j in range(D // 16):
                        sl = pl.ds(j * 16, 16)
                        rows_v[g + k, sl] = rows_v[g + k, sl] * wi

            pltpu.sync_copy(rows_v, acc.at[dst_v], add=True)

        plsc.subcore_barrier()

        @pl.when(sid < N // OWN_ROWS)
        def _():
            base_row = pl.multiple_of(sid * OWN_ROWS, 8)
            pltpu.sync_copy(acc.at[pl.ds(base_row, OWN_ROWS)],
                            out_hbm.at[cid].at[pl.ds(base_row, OWN_ROWS)])

    return agg_kernel(x, src, dst, w)


def _tc_mlp(x, partials, W1x, W1a, b1, W2, b2):
    BLK = 2000

    def body(x_ref, p0_ref, p1_ref, W1x_ref, W1a_ref, b1_ref, W2_ref, b2_ref,
             o_ref):
        agg = p0_ref[0] + p1_ref[0]
        h = jnp.dot(x_ref[...], W1x_ref[...], preferred_element_type=jnp.float32)
        h += jnp.dot(agg, W1a_ref[...], preferred_element_type=jnp.float32)
        h = jnp.maximum(h + b1_ref[...], 0.0)
        o_ref[...] = (
            jnp.dot(h, W2_ref[...], preferred_element_type=jnp.float32)
            + b2_ref[...]
        )

    full = lambda i: (0, 0)
    return pl.pallas_call(
        body,
        grid=(N // BLK,),
        in_specs=[
            pl.BlockSpec((BLK, D), lambda i: (i, 0)),
            pl.BlockSpec((1, BLK, D), lambda i: (0, i, 0)),
            pl.BlockSpec((1, BLK, D), lambda i: (1, i, 0)),
            pl.BlockSpec((D, D), full),
            pl.BlockSpec((D, D), full),
            pl.BlockSpec((1, D), full),
            pl.BlockSpec((D, D), full),
            pl.BlockSpec((1, D), full),
        ],
        out_specs=pl.BlockSpec((BLK, D), lambda i: (i, 0)),
        out_shape=jax.ShapeDtypeStruct((N, D), jnp.float32),
    )(x, partials, partials, W1x, W1a, b1, W2, b2)


def kernel(x, edge_index, edge_weight, W1, b1, W2, b2):
    src = edge_index[1].astype(jnp.int32)
    dst = edge_index[0].astype(jnp.int32)
    half = E // NUM_CORES
    zi = jnp.zeros((PAD,), jnp.int32)
    zf = jnp.zeros((PAD,), jnp.float32)
    src_p = jnp.concatenate([src[:half], zi, src[half:], zi])
    dst_p = jnp.concatenate([dst[:half], zi, dst[half:], zi])
    w_p = jnp.concatenate([edge_weight[:half], zf, edge_weight[half:], zf])
    partials = _sc_aggregate(x, src_p, dst_p, w_p)
    W1x = W1[:D]
    W1a = W1[D:]
    return _tc_mlp(x, partials, W1x, W1a, b1.reshape(1, D), W2,
                   b2.reshape(1, D))


# speedup vs baseline: 1.9595x; 1.9595x over previous
import dataclasses

import jax
import jax.numpy as jnp
from jax import lax
from jax.experimental import pallas as pl
from jax.experimental.pallas import tpu as pltpu
from jax.experimental.pallas import tpu_sc as plsc

N = 10000
D = 128
E = 320000

NUM_CORES = 2
NUM_SUBCORES = 16
CHUNK = 320
NCH = 32
EPC_PAD = NUM_SUBCORES * NCH * CHUNK
PAD = EPC_PAD - E // NUM_CORES
OWN_ROWS = 1000
ZROWS = 40


def _sc_aggregate(x, src, dst, w):
    mesh = plsc.VectorSubcoreMesh(core_axis_name="c", subcore_axis_name="s")

    @pl.kernel(
        out_type=jax.ShapeDtypeStruct((NUM_CORES, N, D), jnp.float32),
        mesh=mesh,
        scratch_types=[
            pltpu.VMEM_SHARED((N, D), jnp.float32),
            pltpu.VMEM((CHUNK, D), jnp.float32),
            pltpu.VMEM((CHUNK,), jnp.int32),
            pltpu.VMEM((CHUNK,), jnp.int32),
            pltpu.VMEM((CHUNK,), jnp.float32),
            pltpu.SemaphoreType.DMA,
        ],
    )
    def agg_kernel(x_hbm, src_hbm, dst_hbm, w_hbm, out_hbm,
                   acc, rows_v, src_v, dst_v, w_v, sem0):
        cid = lax.axis_index("c")
        sid = lax.axis_index("s")

        @pl.when(sid < N // OWN_ROWS)
        def _():
            zero16 = jnp.zeros((16,), jnp.float32)
            for r in range(ZROWS):
                for j in range(D // 16):
                    rows_v[r, pl.ds(j * 16, 16)] = zero16
            base_row = pl.multiple_of(sid * OWN_ROWS, 8)

            @pl.loop(0, OWN_ROWS, step=ZROWS)
            def _(t):
                pltpu.sync_copy(rows_v.at[pl.ds(0, ZROWS)],
                                acc.at[pl.ds(base_row + t, ZROWS)])

        plsc.subcore_barrier()

        cbase = cid * EPC_PAD

        @pl.loop(sid, NCH * NUM_SUBCORES, step=NUM_SUBCORES)
        def _(c):
            b = pl.multiple_of(cbase + c * CHUNK, 8)
            pltpu.sync_copy(src_hbm.at[pl.ds(b, CHUNK)], src_v)
            pltpu.sync_copy(dst_hbm.at[pl.ds(b, CHUNK)], dst_v)
            pltpu.sync_copy(w_hbm.at[pl.ds(b, CHUNK)], w_v)
            pltpu.async_copy(x_hbm.at[src_v], rows_v, sem0).wait()

            @pl.loop(0, CHUNK, step=16)
            def _(g):
                wg = w_v[pl.ds(g, 16)]
                for k in range(16):
                    wi = wg[k]
                    for j in range(D // 16):
                        sl = pl.ds(j * 16, 16)
                        rows_v[g + k, sl] = rows_v[g + k, sl] * wi

            pltpu.sync_copy(rows_v, acc.at[dst_v], add=True)

        plsc.subcore_barrier()

        @pl.when(sid < N // OWN_ROWS)
        def _():
            base_row = pl.multiple_of(sid * OWN_ROWS, 8)
            pltpu.sync_copy(acc.at[pl.ds(base_row, OWN_ROWS)],
                            out_hbm.at[cid].at[pl.ds(base_row, OWN_ROWS)])

    return agg_kernel(x, src, dst, w)


def _tc_mlp(x, partials, W1x, W1a, b1, W2, b2):
    BLK = 2000

    def body(x_ref, p0_ref, p1_ref, W1x_ref, W1a_ref, b1_ref, W2_ref, b2_ref,
             o_ref):
        agg = p0_ref[0] + p1_ref[0]
        h = jnp.dot(x_ref[...], W1x_ref[...], preferred_element_type=jnp.float32)
        h += jnp.dot(agg, W1a_ref[...], preferred_element_type=jnp.float32)
        h = jnp.maximum(h + b1_ref[...], 0.0)
        o_ref[...] = (
            jnp.dot(h, W2_ref[...], preferred_element_type=jnp.float32)
            + b2_ref[...]
        )

    full = lambda i: (0, 0)
    return pl.pallas_call(
        body,
        grid=(N // BLK,),
        in_specs=[
            pl.BlockSpec((BLK, D), lambda i: (i, 0)),
            pl.BlockSpec((1, BLK, D), lambda i: (0, i, 0)),
            pl.BlockSpec((1, BLK, D), lambda i: (1, i, 0)),
            pl.BlockSpec((D, D), full),
            pl.BlockSpec((D, D), full),
            pl.BlockSpec((1, D), full),
            pl.BlockSpec((D, D), full),
            pl.BlockSpec((1, D), full),
        ],
        out_specs=pl.BlockSpec((BLK, D), lambda i: (i, 0)),
        out_shape=jax.ShapeDtypeStruct((N, D), jnp.float32),
    )(x, partials, partials, W1x, W1a, b1, W2, b2)


def kernel(x, edge_index, edge_weight, W1, b1, W2, b2):
    src = edge_index[1].astype(jnp.int32)
    dst = edge_index[0].astype(jnp.int32)
    half = E // NUM_CORES
    zi = jnp.arange(PAD, dtype=jnp.int32)
    zf = jnp.zeros((PAD,), jnp.float32)
    src_p = jnp.concatenate([src[:half], zi, src[half:], zi])
    dst_p = jnp.concatenate([dst[:half], zi, dst[half:], zi])
    w_p = jnp.concatenate([edge_weight[:half], zf, edge_weight[half:], zf])
    partials = _sc_aggregate(x, src_p, dst_p, w_p)
    W1x = W1[:D]
    W1a = W1[D:]
    return _tc_mlp(x, partials, W1x, W1a, b1.reshape(1, D), W2,
                   b2.reshape(1, D))


# speedup vs baseline: 3.0584x; 1.5608x over previous
import dataclasses

import jax
import jax.numpy as jnp
from jax import lax
from jax.experimental import pallas as pl
from jax.experimental.pallas import tpu as pltpu
from jax.experimental.pallas import tpu_sc as plsc

N = 10000
D = 128
E = 320000

NUM_CORES = 2
NUM_SUBCORES = 16
CHUNK = 320
HCHUNK = CHUNK // 2
NCH = 32
EPC_PAD = NUM_SUBCORES * NCH * CHUNK
PAD = EPC_PAD - E // NUM_CORES
OWN_ROWS = 1000
ZROWS = 40


def _sc_aggregate(x, src, dst, w):
    mesh = plsc.VectorSubcoreMesh(core_axis_name="c", subcore_axis_name="s")

    @pl.kernel(
        out_type=jax.ShapeDtypeStruct((NUM_CORES, N, D), jnp.float32),
        mesh=mesh,
        scratch_types=[
            pltpu.VMEM_SHARED((N, D), jnp.float32),
            pltpu.VMEM((HCHUNK, D), jnp.float32),
            pltpu.VMEM((HCHUNK, D), jnp.float32),
        ] + [
            pltpu.VMEM((HCHUNK,), jnp.int32)
            for _ in range(8)
        ] + [
            pltpu.VMEM((HCHUNK,), jnp.float32)
            for _ in range(4)
        ] + [
            pltpu.SemaphoreType.DMA,
            pltpu.SemaphoreType.DMA,
            pltpu.SemaphoreType.DMA,
        ],
    )
    def agg_kernel(x_hbm, src_hbm, dst_hbm, w_hbm, out_hbm,
                   acc, rowsA, rowsB,
                   srcA0, dstA0, srcB0, dstB0, srcA1, dstA1, srcB1, dstB1,
                   wA0, wB0, wA1, wB1,
                   gA, gB, semI):
        cid = lax.axis_index("c")
        sid = lax.axis_index("s")
        rows_v = rowsA

        @pl.when(sid < N // OWN_ROWS)
        def _():
            zero16 = jnp.zeros((16,), jnp.float32)
            for r in range(ZROWS):
                for j in range(D // 16):
                    rows_v[r, pl.ds(j * 16, 16)] = zero16
            base_row = pl.multiple_of(sid * OWN_ROWS, 8)

            @pl.loop(0, OWN_ROWS, step=ZROWS)
            def _(t):
                pltpu.sync_copy(rows_v.at[pl.ds(0, ZROWS)],
                                acc.at[pl.ds(base_row + t, ZROWS)])

        plsc.subcore_barrier()

        ebase = pl.multiple_of(cid * EPC_PAD + sid * (NCH * CHUNK), 8)
        idx_bufs = (
            ((srcA0, dstA0, wA0), (srcB0, dstB0, wB0)),
            ((srcA1, dstA1, wA1), (srcB1, dstB1, wB1)),
        )

        def idx_copies(c, par):
            b = pl.multiple_of(ebase + c * CHUNK, 8)
            (sA, dA, wA), (sB, dB, wB) = idx_bufs[par]
            return (
                pltpu.make_async_copy(src_hbm.at[pl.ds(b, HCHUNK)], sA, semI),
                pltpu.make_async_copy(dst_hbm.at[pl.ds(b, HCHUNK)], dA, semI),
                pltpu.make_async_copy(w_hbm.at[pl.ds(b, HCHUNK)], wA, semI),
                pltpu.make_async_copy(
                    src_hbm.at[pl.ds(b + HCHUNK, HCHUNK)], sB, semI),
                pltpu.make_async_copy(
                    dst_hbm.at[pl.ds(b + HCHUNK, HCHUNK)], dB, semI),
                pltpu.make_async_copy(
                    w_hbm.at[pl.ds(b + HCHUNK, HCHUNK)], wB, semI),
            )

        def scale(rows_h, w_h):
            @pl.loop(0, HCHUNK, step=16)
            def _(g):
                wg = w_h[pl.ds(g, 16)]
                for k in range(16):
                    wi = wg[k]
                    for j in range(D // 16):
                        sl = pl.ds(j * 16, 16)
                        rows_h[g + k, sl] = rows_h[g + k, sl] * wi

        for h in idx_copies(0, 0):
            h.start()
            h.wait()
        pltpu.make_async_copy(x_hbm.at[idx_bufs[0][0][0]], rowsA, gA).start()
        pltpu.make_async_copy(x_hbm.at[idx_bufs[0][1][0]], rowsB, gB).start()

        @pl.loop(0, NCH, step=2)
        def _(k):
            for par in range(2):
                c = k + par
                (sA, dA, wA), (sB, dB, wB) = idx_bufs[par]
                (sAn, _, _), (sBn, _, _) = idx_bufs[1 - par]

                @pl.when(c + 1 < NCH)
                def _():
                    for h in idx_copies(c + 1, 1 - par):
                        h.start()

                pltpu.make_async_copy(x_hbm.at[sA], rowsA, gA).wait()
                scale(rowsA, wA)
                pltpu.sync_copy(rowsA, acc.at[dA], add=True)

                @pl.when(c + 1 < NCH)
                def _():
                    for h in idx_copies(c + 1, 1 - par):
                        h.wait()
                    pltpu.make_async_copy(x_hbm.at[sAn], rowsA, gA).start()

                pltpu.make_async_copy(x_hbm.at[sB], rowsB, gB).wait()
                scale(rowsB, wB)
                pltpu.sync_copy(rowsB, acc.at[dB], add=True)

                @pl.when(c + 1 < NCH)
                def _():
                    pltpu.make_async_copy(x_hbm.at[sBn], rowsB, gB).start()

        plsc.subcore_barrier()

        @pl.when(sid < N // OWN_ROWS)
        def _():
            base_row = pl.multiple_of(sid * OWN_ROWS, 8)
            pltpu.sync_copy(acc.at[pl.ds(base_row, OWN_ROWS)],
                            out_hbm.at[cid].at[pl.ds(base_row, OWN_ROWS)])

    return agg_kernel(x, src, dst, w)


def _tc_mlp(x, partials, W1x, W1a, b1, W2, b2):
    BLK = 2000

    def body(x_ref, p0_ref, p1_ref, W1x_ref, W1a_ref, b1_ref, W2_ref, b2_ref,
             o_ref):
        agg = p0_ref[0] + p1_ref[0]
        h = jnp.dot(x_ref[...], W1x_ref[...], preferred_element_type=jnp.float32)
        h += jnp.dot(agg, W1a_ref[...], preferred_element_type=jnp.float32)
        h = jnp.maximum(h + b1_ref[...], 0.0)
        o_ref[...] = (
            jnp.dot(h, W2_ref[...], preferred_element_type=jnp.float32)
            + b2_ref[...]
        )

    full = lambda i: (0, 0)
    return pl.pallas_call(
        body,
        grid=(N // BLK,),
        in_specs=[
            pl.BlockSpec((BLK, D), lambda i: (i, 0)),
            pl.BlockSpec((1, BLK, D), lambda i: (0, i, 0)),
            pl.BlockSpec((1, BLK, D), lambda i: (1, i, 0)),
            pl.BlockSpec((D, D), full),
            pl.BlockSpec((D, D), full),
            pl.BlockSpec((1, D), full),
            pl.BlockSpec((D, D), full),
            pl.BlockSpec((1, D), full),
        ],
        out_specs=pl.BlockSpec((BLK, D), lambda i: (i, 0)),
        out_shape=jax.ShapeDtypeStruct((N, D), jnp.float32),
    )(x, partials, partials, W1x, W1a, b1, W2, b2)


def kernel(x, edge_index, edge_weight, W1, b1, W2, b2):
    src = edge_index[1].astype(jnp.int32)
    dst = edge_index[0].astype(jnp.int32)
    half = E // NUM_CORES
    zi = jnp.arange(PAD, dtype=jnp.int32)
    zf = jnp.zeros((PAD,), jnp.float32)
    src_p = jnp.concatenate([src[:half], zi, src[half:], zi])
    dst_p = jnp.concatenate([dst[:half], zi, dst[half:], zi])
    w_p = jnp.concatenate([edge_weight[:half], zf, edge_weight[half:], zf])
    partials = _sc_aggregate(x, src_p, dst_p, w_p)
    W1x = W1[:D]
    W1a = W1[D:]
    return _tc_mlp(x, partials, W1x, W1a, b1.reshape(1, D), W2,
                   b2.reshape(1, D))


# speedup vs baseline: 3.0621x; 1.0012x over previous
import dataclasses

import jax
import jax.numpy as jnp
from jax import lax
from jax.experimental import pallas as pl
from jax.experimental.pallas import tpu as pltpu
from jax.experimental.pallas import tpu_sc as plsc

N = 10000
D = 128
E = 320000

NUM_CORES = 2
NUM_SUBCORES = 16
CHUNK = 320
HCHUNK = CHUNK // 2
NCH = 32
EPC_PAD = NUM_SUBCORES * NCH * CHUNK
PAD = EPC_PAD - E // NUM_CORES
OWN_ROWS = 1000
ZROWS = 40


def _sc_aggregate(x, src, dst, w):
    mesh = plsc.VectorSubcoreMesh(core_axis_name="c", subcore_axis_name="s")

    @pl.kernel(
        out_type=jax.ShapeDtypeStruct((NUM_CORES, N, D), jnp.float32),
        mesh=mesh,
        scratch_types=[
            pltpu.VMEM_SHARED((N, D), jnp.float32),
            pltpu.VMEM((HCHUNK, D), jnp.float32),
            pltpu.VMEM((HCHUNK, D), jnp.float32),
        ] + [
            pltpu.VMEM((HCHUNK,), jnp.int32)
            for _ in range(8)
        ] + [
            pltpu.VMEM((HCHUNK,), jnp.float32)
            for _ in range(4)
        ] + [
            pltpu.SemaphoreType.DMA,
            pltpu.SemaphoreType.DMA,
            pltpu.SemaphoreType.DMA,
        ],
    )
    def agg_kernel(x_hbm, src_hbm, dst_hbm, w_hbm, out_hbm,
                   acc, rowsA, rowsB,
                   srcA0, dstA0, srcB0, dstB0, srcA1, dstA1, srcB1, dstB1,
                   wA0, wB0, wA1, wB1,
                   gA, gB, semI):
        cid = lax.axis_index("c")
        sid = lax.axis_index("s")
        rows_v = rowsA

        @pl.when(sid < N // OWN_ROWS)
        def _():
            zero16 = jnp.zeros((16,), jnp.float32)
            for r in range(ZROWS):
                for j in range(D // 16):
                    rows_v[r, pl.ds(j * 16, 16)] = zero16
            base_row = pl.multiple_of(sid * OWN_ROWS, 8)

            @pl.loop(0, OWN_ROWS, step=ZROWS)
            def _(t):
                pltpu.sync_copy(rows_v.at[pl.ds(0, ZROWS)],
                                acc.at[pl.ds(base_row + t, ZROWS)])

        plsc.subcore_barrier()

        ebase = pl.multiple_of(cid * EPC_PAD + sid * (NCH * CHUNK), 8)
        idx_bufs = (
            ((srcA0, dstA0, wA0), (srcB0, dstB0, wB0)),
            ((srcA1, dstA1, wA1), (srcB1, dstB1, wB1)),
        )

        def idx_copies(c, par):
            b = pl.multiple_of(ebase + c * CHUNK, 8)
            (sA, dA, wA), (sB, dB, wB) = idx_bufs[par]
            return (
                pltpu.make_async_copy(src_hbm.at[pl.ds(b, HCHUNK)], sA, semI),
                pltpu.make_async_copy(dst_hbm.at[pl.ds(b, HCHUNK)], dA, semI),
                pltpu.make_async_copy(w_hbm.at[pl.ds(b, HCHUNK)], wA, semI),
                pltpu.make_async_copy(
                    src_hbm.at[pl.ds(b + HCHUNK, HCHUNK)], sB, semI),
                pltpu.make_async_copy(
                    dst_hbm.at[pl.ds(b + HCHUNK, HCHUNK)], dB, semI),
                pltpu.make_async_copy(
                    w_hbm.at[pl.ds(b + HCHUNK, HCHUNK)], wB, semI),
            )

        def scale(rows_h, w_h):
            @plsc.parallel_loop(0, HCHUNK, step=16, unroll=2)
            def _(g):
                wg = w_h[pl.ds(g, 16)]
                for k in range(16):
                    wi = wg[k]
                    for j in range(D // 16):
                        sl = pl.ds(j * 16, 16)
                        rows_h[g + k, sl] = rows_h[g + k, sl] * wi

        for h in idx_copies(0, 0):
            h.start()
            h.wait()
        pltpu.make_async_copy(x_hbm.at[idx_bufs[0][0][0]], rowsA, gA).start()
        pltpu.make_async_copy(x_hbm.at[idx_bufs[0][1][0]], rowsB, gB).start()

        @pl.loop(0, NCH, step=2)
        def _(k):
            for par in range(2):
                c = k + par
                (sA, dA, wA), (sB, dB, wB) = idx_bufs[par]
                (sAn, _, _), (sBn, _, _) = idx_bufs[1 - par]

                @pl.when(c + 1 < NCH)
                def _():
                    for h in idx_copies(c + 1, 1 - par):
                        h.start()

                pltpu.make_async_copy(x_hbm.at[sA], rowsA, gA).wait()
                scale(rowsA, wA)
                pltpu.sync_copy(rowsA, acc.at[dA], add=True)

                @pl.when(c + 1 < NCH)
                def _():
                    for h in idx_copies(c + 1, 1 - par):
                        h.wait()
                    pltpu.make_async_copy(x_hbm.at[sAn], rowsA, gA).start()

                pltpu.make_async_copy(x_hbm.at[sB], rowsB, gB).wait()
                scale(rowsB, wB)
                pltpu.sync_copy(rowsB, acc.at[dB], add=True)

                @pl.when(c + 1 < NCH)
                def _():
                    pltpu.make_async_copy(x_hbm.at[sBn], rowsB, gB).start()

        plsc.subcore_barrier()

        @pl.when(sid < N // OWN_ROWS)
        def _():
            base_row = pl.multiple_of(sid * OWN_ROWS, 8)
            pltpu.sync_copy(acc.at[pl.ds(base_row, OWN_ROWS)],
                            out_hbm.at[cid].at[pl.ds(base_row, OWN_ROWS)])

    return agg_kernel(x, src, dst, w)


def _tc_mlp(x, partials, W1x, W1a, b1, W2, b2):
    BLK = 2000

    def body(x_ref, p0_ref, p1_ref, W1x_ref, W1a_ref, b1_ref, W2_ref, b2_ref,
             o_ref):
        agg = p0_ref[0] + p1_ref[0]
        h = jnp.dot(x_ref[...], W1x_ref[...], preferred_element_type=jnp.float32)
        h += jnp.dot(agg, W1a_ref[...], preferred_element_type=jnp.float32)
        h = jnp.maximum(h + b1_ref[...], 0.0)
        o_ref[...] = (
            jnp.dot(h, W2_ref[...], preferred_element_type=jnp.float32)
            + b2_ref[...]
        )

    full = lambda i: (0, 0)
    return pl.pallas_call(
        body,
        grid=(N // BLK,),
        in_specs=[
            pl.BlockSpec((BLK, D), lambda i: (i, 0)),
            pl.BlockSpec((1, BLK, D), lambda i: (0, i, 0)),
            pl.BlockSpec((1, BLK, D), lambda i: (1, i, 0)),
            pl.BlockSpec((D, D), full),
            pl.BlockSpec((D, D), full),
            pl.BlockSpec((1, D), full),
            pl.BlockSpec((D, D), full),
            pl.BlockSpec((1, D), full),
        ],
        out_specs=pl.BlockSpec((BLK, D), lambda i: (i, 0)),
        out_shape=jax.ShapeDtypeStruct((N, D), jnp.float32),
    )(x, partials, partials, W1x, W1a, b1, W2, b2)


def kernel(x, edge_index, edge_weight, W1, b1, W2, b2):
    src = edge_index[1].astype(jnp.int32)
    dst = edge_index[0].astype(jnp.int32)
    half = E // NUM_CORES
    zi = jnp.arange(PAD, dtype=jnp.int32)
    zf = jnp.zeros((PAD,), jnp.float32)
    src_p = jnp.concatenate([src[:half], zi, src[half:], zi])
    dst_p = jnp.concatenate([dst[:half], zi, dst[half:], zi])
    w_p = jnp.concatenate([edge_weight[:half], zf, edge_weight[half:], zf])
    partials = _sc_aggregate(x, src_p, dst_p, w_p)
    W1x = W1[:D]
    W1a = W1[D:]
    return _tc_mlp(x, partials, W1x, W1a, b1.reshape(1, D), W2,
                   b2.reshape(1, D))
